# Initial kernel scaffold; baseline (speedup 1.0000x reference)
import jax
import jax.numpy as jnp
from jax.experimental import pallas as pl


def kernel(x, router_w, router_b, noisy_w, noisy_b, w1, b1, w2, b2):
    raise NotImplementedError("write your pallas kernel here")


# speedup vs baseline: 1.1733x; 1.1733x over previous
import functools

import jax
import jax.numpy as jnp
from jax.experimental import pallas as pl
from jax.experimental.pallas import tpu as pltpu

E = 8
K = 2
TB = 1024
HB = 1024


def _moe_block(x_ref, rw_ref, rb_ref, w1_ref, b1_ref, w2_ref, b2_ref, o_ref):
    e = pl.program_id(1)
    h = pl.program_id(2)
    xb = x_ref[...]

    logits = (
        jax.lax.dot_general(
            xb, rw_ref[...], (((1,), (1,)), ((), ())),
            preferred_element_type=jnp.float32,
        )
        + rb_ref[...][None, :]
    )
    m1 = jnp.max(logits, axis=-1)
    a1 = jnp.argmax(logits, axis=-1)
    cols = jax.lax.broadcasted_iota(jnp.int32, logits.shape, 1)
    masked = jnp.where(cols == a1[:, None], -jnp.inf, logits)
    m2 = jnp.max(masked, axis=-1)
    a2 = jnp.argmax(masked, axis=-1)
    e2 = jnp.exp(m2 - m1)
    denom = 1.0 + e2
    gate = jnp.where(a1 == e, 1.0 / denom, jnp.where(a2 == e, e2 / denom, 0.0))

    hid = jax.lax.dot_general(
        xb, w1_ref[0], (((1,), (1,)), ((), ())),
        preferred_element_type=jnp.float32,
    ) + b1_ref[0]
    hid = jnp.maximum(hid, 0.0)
    oe = jax.lax.dot_general(
        hid, w2_ref[0], (((1,), (1,)), ((), ())),
        preferred_element_type=jnp.float32,
    )
    oe = jnp.where(h == 0, oe + b2_ref[0], oe)
    contrib = oe * gate[:, None]

    @pl.when((e == 0) & (h == 0))
    def _():
        o_ref[...] = contrib

    @pl.when((e > 0) | (h > 0))
    def _():
        o_ref[...] += contrib


def kernel(x, router_w, router_b, noisy_w, noisy_b, w1, b1, w2, b2):
    del noisy_w, noisy_b
    T, D = x.shape
    H = w1.shape[1]
    b1 = b1.reshape(E, 1, H)
    b2 = b2.reshape(E, 1, D)
    grid = (T // TB, E, H // HB)
    return pl.pallas_call(
        _moe_block,
        grid=grid,
        in_specs=[
            pl.BlockSpec((TB, D), lambda t, e, h: (t, 0)),
            pl.BlockSpec((E, D), lambda t, e, h: (0, 0)),
            pl.BlockSpec((E,), lambda t, e, h: (0,)),
            pl.BlockSpec((1, HB, D), lambda t, e, h: (e, h, 0)),
            pl.BlockSpec((1, 1, HB), lambda t, e, h: (e, 0, h)),
            pl.BlockSpec((1, D, HB), lambda t, e, h: (e, 0, h)),
            pl.BlockSpec((1, 1, D), lambda t, e, h: (e, 0, 0)),
        ],
        out_specs=pl.BlockSpec((TB, D), lambda t, e, h: (t, 0)),
        out_shape=jax.ShapeDtypeStruct((T, D), jnp.float32),
    )(x, router_w, router_b, w1, b1, w2, b2)


# speedup vs baseline: 1.4799x; 1.2613x over previous
import functools

import jax
import jax.numpy as jnp
from jax.experimental import pallas as pl
from jax.experimental.pallas import tpu as pltpu

E = 8
K = 2
T = 4096
D = 1024
H = 4096
N = T * K
B = 256
SLOTS = N + E * B
G = SLOTS // B
JMAX = N // B + 1


def _router_body(x_ref, rw_ref, rb_ref, idx_ref, gate_ref):
    xb = x_ref[...]
    logits = (
        jax.lax.dot_general(
            xb, rw_ref[...], (((1,), (1,)), ((), ())),
            preferred_element_type=jnp.float32,
        )
        + rb_ref[...][None, :]
    )
    m1 = jnp.max(logits, axis=-1)
    a1 = jnp.argmax(logits, axis=-1).astype(jnp.int32)
    cols = jax.lax.broadcasted_iota(jnp.int32, logits.shape, 1)
    masked = jnp.where(cols == a1[:, None], -jnp.inf, logits)
    m2 = jnp.max(masked, axis=-1)
    a2 = jnp.argmax(masked, axis=-1).astype(jnp.int32)
    e2 = jnp.exp(m2 - m1)
    denom = 1.0 + e2
    idx_ref[...] = jnp.stack([a1, a2], axis=-1)
    gate_ref[...] = jnp.stack([1.0 / denom, e2 / denom], axis=-1)


def _router(x, router_w, router_b):
    return pl.pallas_call(
        _router_body,
        grid=(4,),
        in_specs=[
            pl.BlockSpec((T // 4, D), lambda t: (t, 0)),
            pl.BlockSpec((E, D), lambda t: (0, 0)),
            pl.BlockSpec((E,), lambda t: (0,)),
        ],
        out_specs=[
            pl.BlockSpec((T // 4, K), lambda t: (t, 0)),
            pl.BlockSpec((T // 4, K), lambda t: (t, 0)),
        ],
        out_shape=[
            jax.ShapeDtypeStruct((T, K), jnp.int32),
            jax.ShapeDtypeStruct((T, K), jnp.float32),
        ],
    )(x, router_w, router_b)


def _dispatch(topk_idx, x):
    ea = topk_idx.reshape(-1)
    onehot = (ea[:, None] == jnp.arange(E)[None, :]).astype(jnp.int32)
    counts = jnp.sum(onehot, axis=0)
    padded = ((counts + B - 1) // B) * B
    poff = jnp.concatenate([jnp.zeros((1,), jnp.int32),
                            jnp.cumsum(padded)[:-1].astype(jnp.int32)])
    rank = jnp.take_along_axis(jnp.cumsum(onehot, axis=0) - onehot,
                               ea[:, None], axis=1)[:, 0]
    pos = poff[ea] + rank
    tos = jnp.zeros((SLOTS,), jnp.int32).at[pos].set(
        jnp.arange(N, dtype=jnp.int32) // K)
    xs = x[tos]
    nblk = (padded // B).astype(jnp.int32)
    poffb = (poff // B).astype(jnp.int32)
    return pos.reshape(T, K), xs, nblk, poffb


def _ffn_body(nblk_ref, poffb_ref, xs_ref, w1_hbm, b1_ref, w2_hbm, b2_ref,
              o_ref, w1s, w2s, sem1, sem2):
    e = pl.program_id(0)
    j = pl.program_id(1)

    @pl.when((j == 0) & (nblk_ref[e] > 0))
    def _():
        cp1 = pltpu.make_async_copy(w1_hbm.at[e], w1s, sem1)
        cp2 = pltpu.make_async_copy(w2_hbm.at[e], w2s, sem2)
        cp1.start()
        cp2.start()
        cp1.wait()
        cp2.wait()

    @pl.when(j < nblk_ref[e])
    def _():
        xb = xs_ref[...]
        hid = jax.lax.dot_general(
            xb, w1s[...], (((1,), (1,)), ((), ())),
            preferred_element_type=jnp.float32,
        ) + b1_ref[0]
        hid = jnp.maximum(hid, 0.0)
        o_ref[...] = jax.lax.dot_general(
            hid, w2s[...], (((1,), (1,)), ((), ())),
            preferred_element_type=jnp.float32,
        ) + b2_ref[0]


def _ffn(xs, nblk, poffb, w1, b1, w2, b2):
    grid_spec = pltpu.PrefetchScalarGridSpec(
        num_scalar_prefetch=2,
        grid=(E, JMAX),
        in_specs=[
            pl.BlockSpec(
                (B, D),
                lambda e, j, nblk, poffb: (jnp.minimum(poffb[e] + j, G - 1), 0),
            ),
            pl.BlockSpec(memory_space=pl.ANY),
            pl.BlockSpec((1, 1, H), lambda e, j, nblk, poffb: (e, 0, 0)),
            pl.BlockSpec(memory_space=pl.ANY),
            pl.BlockSpec((1, 1, D), lambda e, j, nblk, poffb: (e, 0, 0)),
        ],
        out_specs=pl.BlockSpec(
            (B, D),
            lambda e, j, nblk, poffb: (
                jnp.where(j < nblk[e], poffb[e] + j, G), 0),
        ),
        scratch_shapes=[
            pltpu.VMEM((H, D), jnp.float32),
            pltpu.VMEM((D, H), jnp.float32),
            pltpu.SemaphoreType.DMA,
            pltpu.SemaphoreType.DMA,
        ],
    )
    y = pl.pallas_call(
        _ffn_body,
        grid_spec=grid_spec,
        out_shape=jax.ShapeDtypeStruct((SLOTS + B, D), jnp.float32),
    )(nblk, poffb, xs, w1, b1.reshape(E, 1, H), w2, b2.reshape(E, 1, D))
    return y


def _combine(y, pos, gates):
    return (gates[:, 0:1] * y[pos[:, 0]] + gates[:, 1:2] * y[pos[:, 1]])


def kernel(x, router_w, router_b, noisy_w, noisy_b, w1, b1, w2, b2):
    del noisy_w, noisy_b
    topk_idx, gates = _router(x, router_w, router_b)
    pos, xs, nblk, poffb = _dispatch(topk_idx, x)
    y = _ffn(xs, nblk, poffb, w1, b1, w2, b2)
    return _combine(y, pos, gates)
